# Initial kernel scaffold; baseline (speedup 1.0000x reference)
import jax
import jax.numpy as jnp
from jax.experimental import pallas as pl


def kernel(rule_nodes, rule_weights, point_rule_indices):
    raise NotImplementedError("write your pallas kernel here")


# speedup vs baseline: 300.8461x; 300.8461x over previous
import functools

import jax
import jax.numpy as jnp
from jax import lax
from jax.experimental import pallas as pl
from jax.experimental.pallas import tpu as pltpu
from jax.experimental.pallas import tpu_sc as plsc

_R = 2048
_P = 2_000_000
_D = 8
_L = 16
_NC = 2
_NS = 16
_NW = _NC * _NS
_GROUPS = _P // _L
_GPW = _GROUPS // _NW
_TAIL_PTS = _P - _GPW * _NW * _L
_TPW = _TAIL_PTS // _NW
_GELEMS = _L * _D
_K = 126
_NCHUNK = _GPW // _K
_CH = _K * _GELEMS


def _tables_body(nodes_ref, wts_ref, cr_ref, ci_ref):
    n = nodes_ref[...]
    w = wts_ref[...]
    for d in range(_D):
        ang = n * ((d + 1) / _D)
        cr_ref[d] = w * jnp.cos(ang)
        ci_ref[d] = w * jnp.sin(ang)


_tables = pl.pallas_call(
    _tables_body,
    out_shape=[
        jax.ShapeDtypeStruct((_D, 16, 128), jnp.float32),
        jax.ShapeDtypeStruct((_D, 16, 128), jnp.float32),
    ],
)


def _sc_body(cr_hbm, ci_hbm, idx_hbm, out_hbm,
             cr_v, ci_v, buf0_v, buf1_v, tail_v, acc_v, sem0, sem1):
    wid = lax.axis_index("s") * _NC + lax.axis_index("c")
    pltpu.sync_copy(cr_hbm, cr_v)
    pltpu.sync_copy(ci_hbm, ci_v)

    base = wid * (_GPW * _GELEMS)
    sems = (sem0, sem1)
    bufs = (buf0_v, buf1_v)
    copies = [None, None]
    copies[0] = pltpu.async_copy(idx_hbm.at[pl.ds(base, _CH)], buf0_v, sem0)

    iotas = [lax.iota(jnp.int32, _L) * _D + d for d in range(_D)]

    def group_body(bufref, g, acc):
        gb = g * _GELEMS
        re = im = None
        for d in range(_D):
            vals = plsc.load_gather(bufref, [iotas[d] + gb])
            if d:
                vals = vals + (d * _R)
            c = plsc.load_gather(cr_v, [vals])
            s = plsc.load_gather(ci_v, [vals])
            if re is None:
                re, im = c, s
            else:
                re, im = re * c - im * s, re * s + im * c
        return acc - re

    acc = jnp.zeros((_L,), jnp.float32)
    for ch in range(_NCHUNK):
        nxt = ch + 1
        if nxt < _NCHUNK:
            copies[nxt % 2] = pltpu.async_copy(
                idx_hbm.at[pl.ds(base + nxt * _CH, _CH)],
                bufs[nxt % 2], sems[nxt % 2])
        copies[ch % 2].wait()
        bref = bufs[ch % 2]
        acc = lax.fori_loop(0, _K, lambda g, a: group_body(bref, g, a), acc)

    zi = jnp.zeros((_L,), jnp.int32)
    for i in range(_GELEMS // _L):
        tail_v[pl.ds(i * _L, _L)] = zi
    toff = _GPW * _NW * _GELEMS + wid * (_TPW * _D)
    pltpu.sync_copy(idx_hbm.at[pl.ds(toff, _TPW * _D)],
                    tail_v.at[pl.ds(0, _TPW * _D)])
    tacc = group_body(tail_v, 0, jnp.zeros((_L,), jnp.float32))
    valid = lax.iota(jnp.int32, _L) < _TPW
    acc = acc + jnp.where(valid, tacc, 0.0)

    acc_v[...] = acc
    pltpu.sync_copy(acc_v, out_hbm.at[wid])


_sc_compute = functools.partial(
    pl.kernel,
    out_type=jax.ShapeDtypeStruct((_NW, _L), jnp.float32),
    mesh=plsc.VectorSubcoreMesh(core_axis_name="c", subcore_axis_name="s"),
    compiler_params=pltpu.CompilerParams(needs_layout_passes=False),
    scratch_types=[
        pltpu.VMEM((_D * _R,), jnp.float32),
        pltpu.VMEM((_D * _R,), jnp.float32),
        pltpu.VMEM((_CH,), jnp.int32),
        pltpu.VMEM((_CH,), jnp.int32),
        pltpu.VMEM((_GELEMS,), jnp.int32),
        pltpu.VMEM((_L,), jnp.float32),
        pltpu.SemaphoreType.DMA,
        pltpu.SemaphoreType.DMA,
    ],
)(_sc_body)


def kernel(rule_nodes, rule_weights, point_rule_indices):
    idx = point_rule_indices.astype(jnp.int32).reshape(-1)
    cr, ci = _tables(rule_nodes.reshape(16, 128), rule_weights.reshape(16, 128))
    parts = _sc_compute(cr.reshape(-1), ci.reshape(-1), idx)
    return jnp.sum(parts)


# speedup vs baseline: 2728.0372x; 9.0679x over previous
import functools

import jax
import jax.numpy as jnp
from jax import lax
from jax.experimental import pallas as pl
from jax.experimental.pallas import tpu as pltpu
from jax.experimental.pallas import tpu_sc as plsc

_R = 2048
_P = 2_000_000
_D = 8
_L = 16
_NC = 2
_NS = 16
_NW = _NC * _NS
_TILE = 128
_NT = _P // _TILE
_TPW = _NT // _NW
_XTRA = _NT - _TPW * _NW
_CT = 8
_NCHUNK = _TPW // _CT
_CP = _CT * _TILE
_GPC = _CP // _L
_TGRP = _TILE // _L


def _tables_body(nodes_ref, wts_ref, cr_ref, ci_ref):
    n = nodes_ref[...]
    w = wts_ref[...]
    for d in range(_D):
        ang = n * ((d + 1) / _D)
        cr_ref[d] = w * jnp.cos(ang)
        ci_ref[d] = w * jnp.sin(ang)


_tables = pl.pallas_call(
    _tables_body,
    out_shape=[
        jax.ShapeDtypeStruct((_D, 16, 128), jnp.float32),
        jax.ShapeDtypeStruct((_D, 16, 128), jnp.float32),
    ],
)


def _cmul(a, b):
    (ar, ai), (br, bi) = a, b
    return (ar * br - ai * bi, ar * bi + ai * br)


def _sc_body(cr_hbm, ci_hbm, idx_hbm, out_hbm,
             cr_v, ci_v, buf0_v, buf1_v, tail_v, acc_v, sem0, sem1):
    wid = lax.axis_index("s") * _NC + lax.axis_index("c")
    pltpu.sync_copy(cr_hbm, cr_v)
    pltpu.sync_copy(ci_hbm, ci_v)

    tile0 = wid * _TPW + jnp.minimum(wid, _XTRA)
    p0 = tile0 * _TILE
    sems = (sem0, sem1)
    bufs = (buf0_v, buf1_v)
    copies = [None, None]
    copies[0] = pltpu.async_copy(idx_hbm.at[:, pl.ds(p0, _CP)], buf0_v, sem0)

    def group_body(bufref, g, acc):
        off = g * _L
        cs = []
        for d in range(_D):
            vals = bufref[d, pl.ds(off, _L)]
            if d:
                vals = vals + (d * _R)
            cs.append((plsc.load_gather(cr_v, [vals]),
                       plsc.load_gather(ci_v, [vals])))
        while len(cs) > 1:
            cs = [_cmul(cs[i], cs[i + 1]) for i in range(0, len(cs), 2)]
        return acc - cs[0][0]

    acc = jnp.zeros((_L,), jnp.float32)
    for ch in range(_NCHUNK):
        nxt = ch + 1
        if nxt < _NCHUNK:
            copies[nxt % 2] = pltpu.async_copy(
                idx_hbm.at[:, pl.ds(p0 + nxt * _CP, _CP)],
                bufs[nxt % 2], sems[nxt % 2])
        copies[ch % 2].wait()
        bref = bufs[ch % 2]
        acc = lax.fori_loop(0, _GPC, lambda g, a: group_body(bref, g, a), acc)

    tp = jnp.minimum(tile0 + _TPW, _NT - 1) * _TILE
    pltpu.sync_copy(idx_hbm.at[:, pl.ds(tp, _TILE)], tail_v)
    tacc = lax.fori_loop(
        0, _TGRP, lambda g, a: group_body(tail_v, g, a),
        jnp.zeros((_L,), jnp.float32))
    acc = acc + jnp.where(wid < _XTRA, tacc, jnp.zeros((_L,), jnp.float32))

    acc_v[...] = acc
    pltpu.sync_copy(acc_v, out_hbm.at[wid])


_sc_compute = functools.partial(
    pl.kernel,
    out_type=jax.ShapeDtypeStruct((_NW, _L), jnp.float32),
    mesh=plsc.VectorSubcoreMesh(core_axis_name="c", subcore_axis_name="s"),
    compiler_params=pltpu.CompilerParams(needs_layout_passes=False),
    scratch_types=[
        pltpu.VMEM((_D * _R,), jnp.float32),
        pltpu.VMEM((_D * _R,), jnp.float32),
        pltpu.VMEM((_D, _CP), jnp.int32),
        pltpu.VMEM((_D, _CP), jnp.int32),
        pltpu.VMEM((_D, _TILE), jnp.int32),
        pltpu.VMEM((_L,), jnp.float32),
        pltpu.SemaphoreType.DMA,
        pltpu.SemaphoreType.DMA,
    ],
)(_sc_body)


def kernel(rule_nodes, rule_weights, point_rule_indices):
    idx_t = point_rule_indices.astype(jnp.int32).T
    cr, ci = _tables(rule_nodes.reshape(16, 128), rule_weights.reshape(16, 128))
    parts = _sc_compute(cr.reshape(-1), ci.reshape(-1), idx_t)
    return jnp.sum(parts)
